# initial kernel scaffold (unmeasured)
import jax
import jax.numpy as jnp
from jax import lax
from jax.experimental import pallas as pl
from jax.experimental.pallas import tpu as pltpu

N_DEV = 8


def kernel(x, w_mat):
    m_per, k = x.shape
    _, n = w_mat.shape
    n_per = n // N_DEV

    def body(x_ref, w_ref, out_ref, send_buf, send_sems, recv_sems):
        my = lax.axis_index("i")

        def block(tgt):
            wb = w_ref[:, pl.ds(tgt * n_per, n_per)]
            y = jnp.dot(x_ref[:, :], wb, preferred_element_type=jnp.float32)
            return y * jax.nn.sigmoid(y)

        out_ref[pl.ds(my * m_per, m_per), :] = block(my)

        rdmas = []
        for s in range(1, N_DEV):
            tgt = lax.rem(my + s, N_DEV)
            send_buf[s - 1, :, :] = block(tgt)
            rdma = pltpu.make_async_remote_copy(
                src_ref=send_buf.at[s - 1],
                dst_ref=out_ref.at[pl.ds(my * m_per, m_per), :],
                send_sem=send_sems.at[s - 1],
                recv_sem=recv_sems.at[s - 1],
                device_id=(tgt,),
                device_id_type=pl.DeviceIdType.MESH,
            )
            rdma.start()
            rdmas.append(rdma)

        for s in range(1, N_DEV):
            src = lax.rem(my - s + N_DEV, N_DEV)
            rdmas[s - 1].wait_send()
            recv = pltpu.make_async_remote_copy(
                src_ref=send_buf.at[s - 1],
                dst_ref=out_ref.at[pl.ds(src * m_per, m_per), :],
                send_sem=send_sems.at[s - 1],
                recv_sem=recv_sems.at[s - 1],
                device_id=(src,),
                device_id_type=pl.DeviceIdType.MESH,
            )
            recv.wait_recv()

    return pl.pallas_call(
        body,
        out_shape=jax.ShapeDtypeStruct((N_DEV * m_per, n_per), jnp.float32),
        in_specs=[
            pl.BlockSpec(memory_space=pltpu.VMEM),
            pl.BlockSpec(memory_space=pltpu.VMEM),
        ],
        out_specs=pl.BlockSpec(memory_space=pltpu.VMEM),
        scratch_shapes=[
            pltpu.VMEM((N_DEV - 1, m_per, n_per), jnp.float32),
            pltpu.SemaphoreType.DMA((N_DEV - 1,)),
            pltpu.SemaphoreType.DMA((N_DEV - 1,)),
        ],
        compiler_params=pltpu.CompilerParams(collective_id=0),
    )(x, w_mat)


# baseline (device time: 63003 ns/iter reference)
import jax
import jax.numpy as jnp
from jax import lax
from jax.experimental import pallas as pl
from jax.experimental.pallas import tpu as pltpu

N_DEV = 8


def kernel(x, w_mat):
    m_per, k = x.shape
    _, n = w_mat.shape
    n_per = n // N_DEV

    def body(x_ref, w_ref, out_ref, send_buf, send_sems, recv_sems):
        my = lax.axis_index("i")

        def block(tgt):
            wb = w_ref[:, pl.ds(tgt * n_per, n_per)]
            y = jnp.dot(x_ref[:, :], wb, preferred_element_type=jnp.float32)
            return y * jax.nn.sigmoid(y)

        out_ref[pl.ds(my * m_per, m_per), :] = block(my)

        rdmas = []
        for s in range(1, N_DEV):
            tgt = lax.rem(my + s, N_DEV)
            send_buf[s - 1, :, :] = block(tgt)
            rdma = pltpu.make_async_remote_copy(
                src_ref=send_buf.at[s - 1],
                dst_ref=out_ref.at[pl.ds(my * m_per, m_per), :],
                send_sem=send_sems.at[s - 1],
                recv_sem=recv_sems.at[s - 1],
                device_id=(tgt,),
                device_id_type=pl.DeviceIdType.MESH,
            )
            rdma.start()
            rdmas.append(rdma)

        for s in range(1, N_DEV):
            src = lax.rem(my - s + N_DEV, N_DEV)
            rdmas[s - 1].wait_send()
            recv = pltpu.make_async_remote_copy(
                src_ref=send_buf.at[s - 1],
                dst_ref=out_ref.at[pl.ds(src * m_per, m_per), :],
                send_sem=send_sems.at[s - 1],
                recv_sem=recv_sems.at[s - 1],
                device_id=(src,),
                device_id_type=pl.DeviceIdType.MESH,
            )
            recv.wait_recv()

    return pl.pallas_call(
        body,
        out_shape=jax.ShapeDtypeStruct((N_DEV * m_per, n_per), jnp.float32),
        in_specs=[
            pl.BlockSpec(memory_space=pltpu.VMEM),
            pl.BlockSpec(memory_space=pltpu.VMEM),
        ],
        out_specs=pl.BlockSpec(memory_space=pltpu.VMEM),
        scratch_shapes=[
            pltpu.VMEM((N_DEV - 1, m_per, n_per), jnp.float32),
            pltpu.SemaphoreType.DMA((N_DEV - 1,)),
            pltpu.SemaphoreType.DMA((N_DEV - 1,)),
        ],
        compiler_params=pltpu.CompilerParams(
            vmem_limit_bytes=100 * 1024 * 1024,
        ),
    )(x, w_mat)
